# Initial kernel scaffold; baseline (speedup 1.0000x reference)
import functools

import jax
import jax.numpy as jnp
from jax import lax
from jax.experimental import pallas as pl
from jax.experimental.pallas import tpu as pltpu
from jax.experimental.pallas import tpu_sc as plsc

N_NODES = 10000
N_EDGES = 160000
N_GRAPHS = 64
NCHUNK = 4
CW = 128
NHID = 512

NSUB = 16
EPS = N_EDGES // NSUB
EBLK = 80
NBLK = EPS // EBLK
STRIPE = N_NODES // NSUB

ROWBLK = 1000
NROWBLK = N_NODES // ROWBLK


def _agg_body(src_hbm, dst_hbm, h0, h1, h2, h3, a0, a1, a2, a3,
              src_v, dst_v, rows_a, rows_b, zero_v, acc_s, sem_a, sem_b):
    cid = lax.axis_index("c")
    sid = lax.axis_index("s")

    pltpu.sync_copy(src_hbm.at[sid], src_v)
    pltpu.sync_copy(dst_hbm.at[sid], dst_v)

    zvec = jnp.zeros((16,), jnp.float32)

    @pl.loop(0, 125)
    def _(r):
        @pl.loop(0, CW // 16)
        def _(c):
            zero_v[r, pl.ds(c * 16, 16)] = zvec

    def process(h_hbm, a_hbm):
        @pl.loop(0, STRIPE // 125)
        def _(t):
            pltpu.sync_copy(zero_v, acc_s.at[pl.ds(sid * STRIPE + t * 125, 125)])

        plsc.subcore_barrier()

        pltpu.async_copy(h_hbm.at[src_v.at[0]], rows_a, sem_a)

        @pl.loop(0, (NBLK - 1) // 2)
        def _(k):
            j = k * 2
            pltpu.async_copy(h_hbm.at[src_v.at[j + 1]], rows_b, sem_b)
            pltpu.make_async_copy(h_hbm.at[src_v.at[j]], rows_a, sem_a).wait()
            pltpu.sync_copy(rows_a, acc_s.at[dst_v.at[j]], add=True)
            pltpu.async_copy(h_hbm.at[src_v.at[j + 2]], rows_a, sem_a)
            pltpu.make_async_copy(h_hbm.at[src_v.at[j + 1]], rows_b, sem_b).wait()
            pltpu.sync_copy(rows_b, acc_s.at[dst_v.at[j + 1]], add=True)

        pltpu.make_async_copy(h_hbm.at[src_v.at[NBLK - 1]], rows_a, sem_a).wait()
        pltpu.sync_copy(rows_a, acc_s.at[dst_v.at[NBLK - 1]], add=True)

        plsc.subcore_barrier()

        pltpu.sync_copy(acc_s.at[pl.ds(sid * STRIPE, STRIPE)],
                        a_hbm.at[pl.ds(sid * STRIPE, STRIPE)])

        plsc.subcore_barrier()

    @pl.when(cid == 0)
    def _():
        process(h0, a0)
        process(h1, a1)

    @pl.when(cid == 1)
    def _():
        process(h2, a2)
        process(h3, a3)


def _agg_sc(src_r, dst_r, hs):
    chunk = jax.ShapeDtypeStruct((N_NODES, CW), jnp.float32)
    mesh = plsc.VectorSubcoreMesh(core_axis_name="c", subcore_axis_name="s")
    k = pl.kernel(
        _agg_body,
        out_type=(chunk,) * NCHUNK,
        mesh=mesh,
        scratch_types=[
            pltpu.VMEM((NBLK, EBLK), jnp.int32),
            pltpu.VMEM((NBLK, EBLK), jnp.int32),
            pltpu.VMEM((EBLK, CW), jnp.float32),
            pltpu.VMEM((EBLK, CW), jnp.float32),
            pltpu.VMEM((125, CW), jnp.float32),
            pltpu.VMEM_SHARED((N_NODES, CW), jnp.float32),
            pltpu.SemaphoreType.DMA,
            pltpu.SemaphoreType.DMA,
        ],
    )
    return k(src_r, dst_r, *hs)


def _pre_body(x_ref, w_ref, b_ref, *outs):
    acc = jnp.dot(x_ref[...], w_ref[...],
                  preferred_element_type=jnp.float32) + b_ref[...]
    for c, o in enumerate(outs):
        o[...] = acc[:, c * CW:(c + 1) * CW]


def _pre(x, W0, b0):
    nf = x.shape[1]
    chunk = jax.ShapeDtypeStruct((N_NODES, CW), jnp.float32)
    return pl.pallas_call(
        _pre_body,
        grid=(NROWBLK,),
        in_specs=[
            pl.BlockSpec((ROWBLK, nf), lambda i: (i, 0)),
            pl.BlockSpec((nf, NHID), lambda i: (0, 0)),
            pl.BlockSpec((1, NHID), lambda i: (0, 0)),
        ],
        out_specs=[pl.BlockSpec((ROWBLK, CW), lambda i: (i, 0))] * NCHUNK,
        out_shape=(chunk,) * NCHUNK,
    )(x, W0, b0.reshape(1, NHID))


def _mlp_body(h0, h1, h2, h3, a0, a1, a2, a3, wa_ref, ba_ref, wb_ref, bb_ref,
              *outs):
    hs = (h0, h1, h2, h3)
    as_ = (a0, a1, a2, a3)
    acc = jnp.zeros((ROWBLK, NHID), jnp.float32) + ba_ref[...]
    for c in range(NCHUNK):
        z = hs[c][...] + as_[c][...]
        acc = acc + jnp.dot(z, wa_ref[c], preferred_element_type=jnp.float32)
    u = jnp.maximum(acc, 0.0)
    v = jnp.dot(u, wb_ref[...], preferred_element_type=jnp.float32) + bb_ref[...]
    v = jnp.maximum(v, 0.0)
    for c, o in enumerate(outs):
        o[...] = v[:, c * CW:(c + 1) * CW]


def _mlp(hs, aggs, Wa_i, ba_i, Wb_i, bb_i):
    chunk = jax.ShapeDtypeStruct((N_NODES, CW), jnp.float32)
    cb = pl.BlockSpec((ROWBLK, CW), lambda i: (i, 0))
    return pl.pallas_call(
        _mlp_body,
        grid=(NROWBLK,),
        in_specs=[cb] * NCHUNK + [cb] * NCHUNK + [
            pl.BlockSpec((NCHUNK, CW, NHID), lambda i: (0, 0, 0)),
            pl.BlockSpec((1, NHID), lambda i: (0, 0)),
            pl.BlockSpec((NHID, NHID), lambda i: (0, 0)),
            pl.BlockSpec((1, NHID), lambda i: (0, 0)),
        ],
        out_specs=[cb] * NCHUNK,
        out_shape=(chunk,) * NCHUNK,
    )(*hs, *aggs, Wa_i.reshape(NCHUNK, CW, NHID), ba_i.reshape(1, NHID),
      Wb_i, bb_i.reshape(1, NHID))


def _pool_body(h0, h1, h2, h3, batch_ref, wp_ref, bp_ref, wr_ref, br_ref,
               out_ref, g_acc):
    i = pl.program_id(0)

    @pl.when(i == 0)
    def _():
        g_acc[...] = jnp.zeros((N_GRAPHS, NHID), jnp.float32)

    b = batch_ref[0, 0, :]
    onehot = (b[:, None] == lax.broadcasted_iota(jnp.int32, (1, N_GRAPHS), 1)
              ).astype(jnp.float32)
    hs = (h0, h1, h2, h3)
    z = jnp.concatenate([hs[c][...] for c in range(NCHUNK)], axis=1)
    g_acc[...] += lax.dot_general(onehot, z, (((0,), (0,)), ((), ())),
                                  preferred_element_type=jnp.float32)

    @pl.when(i == NROWBLK - 1)
    def _():
        g = g_acc[...]
        g = jnp.maximum(jnp.dot(g, wp_ref[...],
                                preferred_element_type=jnp.float32)
                        + bp_ref[...], 0.0)
        o = jnp.dot(g, wr_ref[...], preferred_element_type=jnp.float32) \
            + br_ref[...]
        m = jnp.max(o, axis=1, keepdims=True)
        lse = jnp.log(jnp.sum(jnp.exp(o - m), axis=1, keepdims=True)) + m
        out_ref[...] = o - lse


def _pool(hs, batch, Wp, bp, Wr, br):
    nclass = Wr.shape[1]
    cb = pl.BlockSpec((ROWBLK, CW), lambda i: (i, 0))
    return pl.pallas_call(
        _pool_body,
        grid=(NROWBLK,),
        in_specs=[cb] * NCHUNK + [
            pl.BlockSpec((1, 1, ROWBLK), lambda i: (i, 0, 0)),
            pl.BlockSpec((NHID, NHID), lambda i: (0, 0)),
            pl.BlockSpec((1, NHID), lambda i: (0, 0)),
            pl.BlockSpec((NHID, nclass), lambda i: (0, 0)),
            pl.BlockSpec((1, nclass), lambda i: (0, 0)),
        ],
        out_specs=pl.BlockSpec((N_GRAPHS, nclass), lambda i: (0, 0)),
        out_shape=jax.ShapeDtypeStruct((N_GRAPHS, nclass), jnp.float32),
        scratch_shapes=[pltpu.VMEM((N_GRAPHS, NHID), jnp.float32)],
    )(*hs, batch.reshape(NROWBLK, 1, ROWBLK).astype(jnp.int32),
      Wp, bp.reshape(1, NHID), Wr, br.reshape(1, nclass))


def kernel(x, edge_index, batch, W0, b0, Wa, ba, Wb, bb, Wp, bp, Wr, br):
    src = edge_index[0].astype(jnp.int32).reshape(NSUB, NBLK, EBLK)
    dst = edge_index[1].astype(jnp.int32).reshape(NSUB, NBLK, EBLK)
    hs = _pre(x, W0, b0)
    nlayer = Wa.shape[0]
    for i in range(nlayer):
        aggs = _agg_sc(src, dst, hs)
        hs = _mlp(hs, aggs, Wa[i], ba[i], Wb[i], bb[i])
    return _pool(hs, batch, Wp, bp, Wr, br)


# speedup vs baseline: 3.8554x; 3.8554x over previous
import jax
import jax.numpy as jnp
from jax import lax
from jax.experimental import pallas as pl
from jax.experimental.pallas import tpu as pltpu
from jax.experimental.pallas import tpu_sc as plsc

N_NODES = 10000
N_EDGES = 160000
N_GRAPHS = 64
NCHUNK = 8
CW = 64
NHID = 512

NSUB = 16
NCORE = 2
CPC = NCHUNK // NCORE
EPS = N_EDGES // NSUB
EBLK = 80
NBLK = EPS // EBLK
ZSTRIPE = 1000
NZSUB = N_NODES // ZSTRIPE
ZBUF = 200

ROWBLK = 1000
NROWBLK = N_NODES // ROWBLK


def _agg_body(src_hbm, dst_hbm, *refs):
    hs = refs[:NCHUNK]
    aggs = refs[NCHUNK:2 * NCHUNK]
    (src_v, dst_v, rows_a, rows_b, zero_v, acc_s, sem_a, sem_b) = refs[2 * NCHUNK:]
    cid = lax.axis_index("c")
    sid = lax.axis_index("s")

    pltpu.sync_copy(src_hbm.at[sid], src_v)
    pltpu.sync_copy(dst_hbm.at[sid], dst_v)

    zvec = jnp.zeros((16,), jnp.float32)

    @pl.loop(0, ZBUF)
    def _(r):
        @pl.loop(0, CW // 16)
        def _(c):
            zero_v[r, pl.ds(c * 16, 16)] = zvec

    def process(h_hbm, a_hbm):
        @pl.when(sid < NZSUB)
        def _():
            @pl.loop(0, ZSTRIPE // ZBUF)
            def _(t):
                pltpu.sync_copy(zero_v,
                                acc_s.at[pl.ds(sid * ZSTRIPE + t * ZBUF, ZBUF)])

        plsc.subcore_barrier()

        pltpu.async_copy(h_hbm.at[src_v.at[0]], rows_a, sem_a)

        @pl.loop(0, (NBLK - 1) // 2)
        def _(k):
            j = k * 2
            pltpu.async_copy(h_hbm.at[src_v.at[j + 1]], rows_b, sem_b)
            pltpu.make_async_copy(h_hbm.at[src_v.at[j]], rows_a, sem_a).wait()
            pltpu.sync_copy(rows_a, acc_s.at[dst_v.at[j]], add=True)
            pltpu.async_copy(h_hbm.at[src_v.at[j + 2]], rows_a, sem_a)
            pltpu.make_async_copy(h_hbm.at[src_v.at[j + 1]], rows_b, sem_b).wait()
            pltpu.sync_copy(rows_b, acc_s.at[dst_v.at[j + 1]], add=True)

        pltpu.make_async_copy(h_hbm.at[src_v.at[NBLK - 1]], rows_a, sem_a).wait()
        pltpu.sync_copy(rows_a, acc_s.at[dst_v.at[NBLK - 1]], add=True)

        plsc.subcore_barrier()

        @pl.when(sid < NZSUB)
        def _():
            pltpu.sync_copy(acc_s.at[pl.ds(sid * ZSTRIPE, ZSTRIPE)],
                            a_hbm.at[pl.ds(sid * ZSTRIPE, ZSTRIPE)])

        plsc.subcore_barrier()

    @pl.when(cid == 0)
    def _():
        for c in range(CPC):
            process(hs[c], aggs[c])

    @pl.when(cid == 1)
    def _():
        for c in range(CPC):
            process(hs[CPC + c], aggs[CPC + c])


def _agg_sc(src_r, dst_r, hs):
    chunk = jax.ShapeDtypeStruct((N_NODES, CW), jnp.float32)
    mesh = plsc.VectorSubcoreMesh(core_axis_name="c", subcore_axis_name="s")
    k = pl.kernel(
        _agg_body,
        out_type=(chunk,) * NCHUNK,
        mesh=mesh,
        scratch_types=[
            pltpu.VMEM((NBLK, EBLK), jnp.int32),
            pltpu.VMEM((NBLK, EBLK), jnp.int32),
            pltpu.VMEM((EBLK, CW), jnp.float32),
            pltpu.VMEM((EBLK, CW), jnp.float32),
            pltpu.VMEM((ZBUF, CW), jnp.float32),
            pltpu.VMEM_SHARED((N_NODES, CW), jnp.float32),
            pltpu.SemaphoreType.DMA,
            pltpu.SemaphoreType.DMA,
        ],
        compiler_params=pltpu.CompilerParams(use_tc_tiling_on_sc=False),
    )
    return k(src_r, dst_r, *hs)


def _pre_body(x_ref, w_ref, b_ref, *outs):
    acc = jnp.dot(x_ref[...], w_ref[...],
                  preferred_element_type=jnp.float32) + b_ref[...]
    for c, o in enumerate(outs):
        o[...] = acc[:, c * CW:(c + 1) * CW]


def _pre(x, W0, b0):
    nf = x.shape[1]
    chunk = jax.ShapeDtypeStruct((N_NODES, CW), jnp.float32)
    return pl.pallas_call(
        _pre_body,
        grid=(NROWBLK,),
        in_specs=[
            pl.BlockSpec((ROWBLK, nf), lambda i: (i, 0)),
            pl.BlockSpec((nf, NHID), lambda i: (0, 0)),
            pl.BlockSpec((1, NHID), lambda i: (0, 0)),
        ],
        out_specs=[pl.BlockSpec((ROWBLK, CW), lambda i: (i, 0))] * NCHUNK,
        out_shape=(chunk,) * NCHUNK,
    )(x, W0, b0.reshape(1, NHID))


def _mlp_body(*refs):
    hs = refs[:NCHUNK]
    as_ = refs[NCHUNK:2 * NCHUNK]
    wa_ref, ba_ref, wb_ref, bb_ref = refs[2 * NCHUNK:2 * NCHUNK + 4]
    outs = refs[2 * NCHUNK + 4:]
    z = jnp.concatenate([hs[c][...] + as_[c][...] for c in range(NCHUNK)],
                        axis=1)
    acc = jnp.dot(z, wa_ref[...], preferred_element_type=jnp.float32) \
        + ba_ref[...]
    u = jnp.maximum(acc, 0.0)
    v = jnp.dot(u, wb_ref[...], preferred_element_type=jnp.float32) \
        + bb_ref[...]
    v = jnp.maximum(v, 0.0)
    for c, o in enumerate(outs):
        o[...] = v[:, c * CW:(c + 1) * CW]


def _mlp(hs, aggs, Wa_i, ba_i, Wb_i, bb_i):
    chunk = jax.ShapeDtypeStruct((N_NODES, CW), jnp.float32)
    cb = pl.BlockSpec((ROWBLK, CW), lambda i: (i, 0))
    return pl.pallas_call(
        _mlp_body,
        grid=(NROWBLK,),
        in_specs=[cb] * NCHUNK + [cb] * NCHUNK + [
            pl.BlockSpec((NHID, NHID), lambda i: (0, 0)),
            pl.BlockSpec((1, NHID), lambda i: (0, 0)),
            pl.BlockSpec((NHID, NHID), lambda i: (0, 0)),
            pl.BlockSpec((1, NHID), lambda i: (0, 0)),
        ],
        out_specs=[cb] * NCHUNK,
        out_shape=(chunk,) * NCHUNK,
    )(*hs, *aggs, Wa_i, ba_i.reshape(1, NHID), Wb_i, bb_i.reshape(1, NHID))


def _pool_body(*refs):
    hs = refs[:NCHUNK]
    batch_ref, wp_ref, bp_ref, wr_ref, br_ref = refs[NCHUNK:NCHUNK + 5]
    out_ref, g_acc = refs[NCHUNK + 5:]
    i = pl.program_id(0)

    @pl.when(i == 0)
    def _():
        g_acc[...] = jnp.zeros((N_GRAPHS, NHID), jnp.float32)

    b = batch_ref[0, 0, :]
    onehot = (b[:, None] == lax.broadcasted_iota(jnp.int32, (1, N_GRAPHS), 1)
              ).astype(jnp.float32)
    z = jnp.concatenate([hs[c][...] for c in range(NCHUNK)], axis=1)
    g_acc[...] += lax.dot_general(onehot, z, (((0,), (0,)), ((), ())),
                                  preferred_element_type=jnp.float32)

    @pl.when(i == NROWBLK - 1)
    def _():
        g = g_acc[...]
        g = jnp.maximum(jnp.dot(g, wp_ref[...],
                                preferred_element_type=jnp.float32)
                        + bp_ref[...], 0.0)
        o = jnp.dot(g, wr_ref[...], preferred_element_type=jnp.float32) \
            + br_ref[...]
        m = jnp.max(o, axis=1, keepdims=True)
        lse = jnp.log(jnp.sum(jnp.exp(o - m), axis=1, keepdims=True)) + m
        out_ref[...] = o - lse


def _pool(hs, batch, Wp, bp, Wr, br):
    nclass = Wr.shape[1]
    cb = pl.BlockSpec((ROWBLK, CW), lambda i: (i, 0))
    return pl.pallas_call(
        _pool_body,
        grid=(NROWBLK,),
        in_specs=[cb] * NCHUNK + [
            pl.BlockSpec((1, 1, ROWBLK), lambda i: (i, 0, 0)),
            pl.BlockSpec((NHID, NHID), lambda i: (0, 0)),
            pl.BlockSpec((1, NHID), lambda i: (0, 0)),
            pl.BlockSpec((NHID, nclass), lambda i: (0, 0)),
            pl.BlockSpec((1, nclass), lambda i: (0, 0)),
        ],
        out_specs=pl.BlockSpec((N_GRAPHS, nclass), lambda i: (0, 0)),
        out_shape=jax.ShapeDtypeStruct((N_GRAPHS, nclass), jnp.float32),
        scratch_shapes=[pltpu.VMEM((N_GRAPHS, NHID), jnp.float32)],
    )(*hs, batch.reshape(NROWBLK, 1, ROWBLK).astype(jnp.int32),
      Wp, bp.reshape(1, NHID), Wr, br.reshape(1, nclass))


def kernel(x, edge_index, batch, W0, b0, Wa, ba, Wb, bb, Wp, bp, Wr, br):
    src = edge_index[0].astype(jnp.int32).reshape(NSUB, NBLK, EBLK)
    dst = edge_index[1].astype(jnp.int32).reshape(NSUB, NBLK, EBLK)
    hs = _pre(x, W0, b0)

    def layer(carry, w):
        Wa_i, ba_i, Wb_i, bb_i = w
        aggs = _agg_sc(src, dst, carry)
        return _mlp(carry, aggs, Wa_i, ba_i, Wb_i, bb_i), None

    hs, _ = lax.scan(layer, hs, (Wa, ba, Wb, bb))
    return _pool(hs, batch, Wp, bp, Wr, br)


# speedup vs baseline: 4.6194x; 1.1981x over previous
import jax
import jax.numpy as jnp
from jax import lax
from jax.experimental import pallas as pl
from jax.experimental.pallas import tpu as pltpu
from jax.experimental.pallas import tpu_sc as plsc

N_NODES = 10000
N_EDGES = 160000
N_GRAPHS = 64
NCHUNK = 4
CW = 128
NHID = 512
BF = jnp.bfloat16

NSUB = 16
NCORE = 2
CPC = NCHUNK // NCORE
EBLK = 128
EPS_PAD = 10240
NBLK = EPS_PAD // EBLK
TRASH = N_NODES
ACC_ROWS = N_NODES + 16
ZSTRIPE = 1000
NZSUB = N_NODES // ZSTRIPE
ZBUF = 200

ROWBLK = 2000
NROWBLK = N_NODES // ROWBLK


def _agg_body(src_hbm, dst_hbm, *refs):
    hs = refs[:NCHUNK]
    aggs = refs[NCHUNK:2 * NCHUNK]
    rest = refs[2 * NCHUNK:]
    src_v, dst_v = rest[0], rest[1]
    rows = rest[2:6]
    zero_v, acc_s = rest[6], rest[7]
    gsem = rest[8:12]
    ssem = rest[12:16]
    cid = lax.axis_index("c")
    sid = lax.axis_index("s")

    pltpu.sync_copy(src_hbm.at[sid], src_v)
    pltpu.sync_copy(dst_hbm.at[sid], dst_v)

    zvec = jnp.zeros((32,), BF)

    @pl.loop(0, ZBUF)
    def _(r):
        @pl.loop(0, CW // 32)
        def _(c):
            zero_v[r, pl.ds(c * 32, 32)] = zvec

    def process(h_hbm, a_hbm):
        @pl.when(sid < NZSUB)
        def _():
            @pl.loop(0, ZSTRIPE // ZBUF)
            def _(t):
                pltpu.sync_copy(zero_v,
                                acc_s.at[pl.ds(sid * ZSTRIPE + t * ZBUF, ZBUF)])

        plsc.subcore_barrier()

        for i in range(3):
            pltpu.async_copy(h_hbm.at[src_v.at[i]], rows[i], gsem[i])

        @pl.loop(0, NBLK // 4)
        def _(g):
            for k4 in range(4):
                j = g * 4 + k4
                b3 = (k4 + 3) % 4
                pltpu.make_async_copy(h_hbm.at[src_v.at[j]], rows[k4],
                                      gsem[k4]).wait()
                pltpu.async_copy(rows[k4], acc_s.at[dst_v.at[j]], ssem[k4],
                                 add=True)
                if k4 == 0:
                    @pl.when(g >= 1)
                    def _():
                        pltpu.make_async_copy(rows[b3], acc_s.at[dst_v.at[j]],
                                              ssem[b3]).wait()
                    pltpu.async_copy(h_hbm.at[src_v.at[j + 3]], rows[b3],
                                     gsem[b3])
                else:
                    @pl.when(g <= NBLK // 4 - 2)
                    def _():
                        pltpu.make_async_copy(rows[b3], acc_s.at[dst_v.at[j]],
                                              ssem[b3]).wait()
                        pltpu.async_copy(h_hbm.at[src_v.at[j + 3]], rows[b3],
                                         gsem[b3])

        for k4 in range(4):
            pltpu.make_async_copy(rows[k4], acc_s.at[dst_v.at[0]],
                                  ssem[k4]).wait()

        plsc.subcore_barrier()

        @pl.when(sid < NZSUB)
        def _():
            pltpu.sync_copy(acc_s.at[pl.ds(sid * ZSTRIPE, ZSTRIPE)],
                            a_hbm.at[pl.ds(sid * ZSTRIPE, ZSTRIPE)])

        plsc.subcore_barrier()

    @pl.when(cid == 0)
    def _():
        for c in range(CPC):
            process(hs[c], aggs[c])

    @pl.when(cid == 1)
    def _():
        for c in range(CPC):
            process(hs[CPC + c], aggs[CPC + c])


def _agg_sc(src_r, dst_r, hs):
    chunk = jax.ShapeDtypeStruct((N_NODES, CW), BF)
    mesh = plsc.VectorSubcoreMesh(core_axis_name="c", subcore_axis_name="s")
    k = pl.kernel(
        _agg_body,
        out_type=(chunk,) * NCHUNK,
        mesh=mesh,
        scratch_types=[
            pltpu.VMEM((NBLK, EBLK), jnp.int32),
            pltpu.VMEM((NBLK, EBLK), jnp.int32),
            pltpu.VMEM((EBLK, CW), BF),
            pltpu.VMEM((EBLK, CW), BF),
            pltpu.VMEM((EBLK, CW), BF),
            pltpu.VMEM((EBLK, CW), BF),
            pltpu.VMEM((ZBUF, CW), BF),
            pltpu.VMEM_SHARED((ACC_ROWS, CW), BF),
            pltpu.SemaphoreType.DMA,
            pltpu.SemaphoreType.DMA,
            pltpu.SemaphoreType.DMA,
            pltpu.SemaphoreType.DMA,
            pltpu.SemaphoreType.DMA,
            pltpu.SemaphoreType.DMA,
            pltpu.SemaphoreType.DMA,
            pltpu.SemaphoreType.DMA,
        ],
        compiler_params=pltpu.CompilerParams(use_tc_tiling_on_sc=False),
    )
    return k(src_r, dst_r, *hs)


def _pre_body(x_ref, w_ref, b_ref, *outs):
    acc = jnp.dot(x_ref[...], w_ref[...],
                  preferred_element_type=jnp.float32) + b_ref[...]
    for c, o in enumerate(outs):
        o[...] = acc[:, c * CW:(c + 1) * CW].astype(BF)


def _pre(x, W0, b0):
    nf = x.shape[1]
    chunk = jax.ShapeDtypeStruct((N_NODES, CW), BF)
    return pl.pallas_call(
        _pre_body,
        grid=(NROWBLK,),
        in_specs=[
            pl.BlockSpec((ROWBLK, nf), lambda i: (i, 0)),
            pl.BlockSpec((nf, NHID), lambda i: (0, 0)),
            pl.BlockSpec((1, NHID), lambda i: (0, 0)),
        ],
        out_specs=[pl.BlockSpec((ROWBLK, CW), lambda i: (i, 0))] * NCHUNK,
        out_shape=(chunk,) * NCHUNK,
    )(x, W0, b0.reshape(1, NHID))


def _mlp_body(*refs):
    hs = refs[:NCHUNK]
    as_ = refs[NCHUNK:2 * NCHUNK]
    wa_ref, ba_ref, wb_ref, bb_ref = refs[2 * NCHUNK:2 * NCHUNK + 4]
    outs = refs[2 * NCHUNK + 4:]
    z = jnp.concatenate(
        [hs[c][...].astype(jnp.float32) + as_[c][...].astype(jnp.float32)
         for c in range(NCHUNK)], axis=1)
    acc = jnp.dot(z, wa_ref[...], preferred_element_type=jnp.float32) \
        + ba_ref[...]
    u = jnp.maximum(acc, 0.0)
    v = jnp.dot(u, wb_ref[...], preferred_element_type=jnp.float32) \
        + bb_ref[...]
    v = jnp.maximum(v, 0.0)
    for c, o in enumerate(outs):
        o[...] = v[:, c * CW:(c + 1) * CW].astype(BF)


def _mlp(hs, aggs, Wa_i, ba_i, Wb_i, bb_i):
    chunk = jax.ShapeDtypeStruct((N_NODES, CW), BF)
    cb = pl.BlockSpec((ROWBLK, CW), lambda i: (i, 0))
    return pl.pallas_call(
        _mlp_body,
        grid=(NROWBLK,),
        in_specs=[cb] * NCHUNK + [cb] * NCHUNK + [
            pl.BlockSpec((NHID, NHID), lambda i: (0, 0)),
            pl.BlockSpec((1, NHID), lambda i: (0, 0)),
            pl.BlockSpec((NHID, NHID), lambda i: (0, 0)),
            pl.BlockSpec((1, NHID), lambda i: (0, 0)),
        ],
        out_specs=[cb] * NCHUNK,
        out_shape=(chunk,) * NCHUNK,
    )(*hs, *aggs, Wa_i, ba_i.reshape(1, NHID), Wb_i, bb_i.reshape(1, NHID))


def _pool_body(*refs):
    hs = refs[:NCHUNK]
    batch_ref, wp_ref, bp_ref, wr_ref, br_ref = refs[NCHUNK:NCHUNK + 5]
    out_ref, g_acc = refs[NCHUNK + 5:]
    i = pl.program_id(0)

    @pl.when(i == 0)
    def _():
        g_acc[...] = jnp.zeros((N_GRAPHS, NHID), jnp.float32)

    b = batch_ref[0, 0, :]
    onehot = (b[:, None] == lax.broadcasted_iota(jnp.int32, (1, N_GRAPHS), 1)
              ).astype(jnp.float32)
    z = jnp.concatenate([hs[c][...].astype(jnp.float32)
                         for c in range(NCHUNK)], axis=1)
    g_acc[...] += lax.dot_general(onehot, z, (((0,), (0,)), ((), ())),
                                  preferred_element_type=jnp.float32)

    @pl.when(i == NROWBLK - 1)
    def _():
        g = g_acc[...]
        g = jnp.maximum(jnp.dot(g, wp_ref[...],
                                preferred_element_type=jnp.float32)
                        + bp_ref[...], 0.0)
        o = jnp.dot(g, wr_ref[...], preferred_element_type=jnp.float32) \
            + br_ref[...]
        m = jnp.max(o, axis=1, keepdims=True)
        lse = jnp.log(jnp.sum(jnp.exp(o - m), axis=1, keepdims=True)) + m
        out_ref[...] = o - lse


def _pool(hs, batch, Wp, bp, Wr, br):
    nclass = Wr.shape[1]
    cb = pl.BlockSpec((ROWBLK, CW), lambda i: (i, 0))
    return pl.pallas_call(
        _pool_body,
        grid=(NROWBLK,),
        in_specs=[cb] * NCHUNK + [
            pl.BlockSpec((1, 1, ROWBLK), lambda i: (i, 0, 0)),
            pl.BlockSpec((NHID, NHID), lambda i: (0, 0)),
            pl.BlockSpec((1, NHID), lambda i: (0, 0)),
            pl.BlockSpec((NHID, nclass), lambda i: (0, 0)),
            pl.BlockSpec((1, nclass), lambda i: (0, 0)),
        ],
        out_specs=pl.BlockSpec((N_GRAPHS, nclass), lambda i: (0, 0)),
        out_shape=jax.ShapeDtypeStruct((N_GRAPHS, nclass), jnp.float32),
        scratch_shapes=[pltpu.VMEM((N_GRAPHS, NHID), jnp.float32)],
    )(*hs, batch.reshape(NROWBLK, 1, ROWBLK).astype(jnp.int32),
      Wp, bp.reshape(1, NHID), Wr, br.reshape(1, nclass))


def kernel(x, edge_index, batch, W0, b0, Wa, ba, Wb, bb, Wp, bp, Wr, br):
    eps = N_EDGES // NSUB
    pad = EPS_PAD - eps
    src = jnp.pad(edge_index[0].astype(jnp.int32).reshape(NSUB, eps),
                  ((0, 0), (0, pad))).reshape(NSUB, NBLK, EBLK)
    dst = jnp.pad(edge_index[1].astype(jnp.int32).reshape(NSUB, eps),
                  ((0, 0), (0, pad)),
                  constant_values=TRASH).reshape(NSUB, NBLK, EBLK)
    hs = _pre(x, W0, b0)

    def layer(carry, w):
        Wa_i, ba_i, Wb_i, bb_i = w
        aggs = _agg_sc(src, dst, carry)
        return _mlp(carry, aggs, Wa_i, ba_i, Wb_i, bb_i), None

    hs, _ = lax.scan(layer, hs, (Wa, ba, Wb, bb))
    return _pool(hs, batch, Wp, bp, Wr, br)


# speedup vs baseline: 4.7275x; 1.0234x over previous
import jax
import jax.numpy as jnp
from jax import lax
from jax.experimental import pallas as pl
from jax.experimental.pallas import tpu as pltpu
from jax.experimental.pallas import tpu_sc as plsc

N_NODES = 10000
N_EDGES = 160000
N_GRAPHS = 64
NCHUNK = 4
CW = 128
NHID = 512
BF = jnp.bfloat16

NSUB = 16
NCORE = 2
CPC = NCHUNK // NCORE
EBLK = 128
RING = 5
EPS_PAD = 10240
NBLK = EPS_PAD // EBLK
TRASH = N_NODES
ACC_ROWS = N_NODES + 16
ZSTRIPE = 1000
NZSUB = N_NODES // ZSTRIPE
ZBUF = 200

ROWBLK = 2000
NROWBLK = N_NODES // ROWBLK


def _agg_body(src_hbm, dst_hbm, *refs):
    hs = refs[:NCHUNK]
    aggs = refs[NCHUNK:2 * NCHUNK]
    rest = refs[2 * NCHUNK:]
    src_v, dst_v = rest[0], rest[1]
    rows = rest[2:2 + RING]
    zero_v, acc_s = rest[2 + RING], rest[3 + RING]
    gsem = rest[4 + RING:4 + 2 * RING]
    ssem = rest[4 + 2 * RING:4 + 3 * RING]
    zsem = rest[4 + 3 * RING]
    cid = lax.axis_index("c")
    sid = lax.axis_index("s")

    pltpu.sync_copy(src_hbm.at[sid], src_v)
    pltpu.sync_copy(dst_hbm.at[sid], dst_v)

    zvec = jnp.zeros((32,), BF)

    @pl.loop(0, ZBUF)
    def _(r):
        @pl.loop(0, CW // 32)
        def _(c):
            zero_v[r, pl.ds(c * 32, 32)] = zvec

    def process(h_hbm, a_hbm):
        for i in range(RING - 1):
            pltpu.async_copy(h_hbm.at[src_v.at[i]], rows[i], gsem[i])

        @pl.when(sid < NZSUB)
        def _():
            @pl.loop(0, ZSTRIPE // ZBUF)
            def _(t):
                pltpu.sync_copy(zero_v,
                                acc_s.at[pl.ds(sid * ZSTRIPE + t * ZBUF,
                                               ZBUF)])

        plsc.subcore_barrier()

        @pl.loop(0, NBLK // RING)
        def _(g):
            for kk in range(RING):
                j = g * RING + kk
                bp = (kk + RING - 1) % RING
                pltpu.make_async_copy(h_hbm.at[src_v.at[j]], rows[kk],
                                      gsem[kk]).wait()
                pltpu.async_copy(rows[kk], acc_s.at[dst_v.at[j]], ssem[kk],
                                 add=True)
                if kk == 0:
                    @pl.when(g >= 1)
                    def _():
                        pltpu.make_async_copy(rows[bp], acc_s.at[dst_v.at[j]],
                                              ssem[bp]).wait()
                    pltpu.async_copy(h_hbm.at[src_v.at[j + RING - 1]],
                                     rows[bp], gsem[bp])
                else:
                    @pl.when(g <= NBLK // RING - 2)
                    def _():
                        pltpu.make_async_copy(rows[bp], acc_s.at[dst_v.at[j]],
                                              ssem[bp]).wait()
                        pltpu.async_copy(h_hbm.at[src_v.at[j + RING - 1]],
                                         rows[bp], gsem[bp])

        for kk in range(RING):
            pltpu.make_async_copy(rows[kk], acc_s.at[dst_v.at[0]],
                                  ssem[kk]).wait()

        plsc.subcore_barrier()

        @pl.when(sid < NZSUB)
        def _():
            pltpu.sync_copy(acc_s.at[pl.ds(sid * ZSTRIPE, ZSTRIPE)],
                            a_hbm.at[pl.ds(sid * ZSTRIPE, ZSTRIPE)])

        plsc.subcore_barrier()

    @pl.when(cid == 0)
    def _():
        for c in range(CPC):
            process(hs[c], aggs[c])

    @pl.when(cid == 1)
    def _():
        for c in range(CPC):
            process(hs[CPC + c], aggs[CPC + c])


def _agg_sc(src_r, dst_r, hs):
    chunk = jax.ShapeDtypeStruct((N_NODES, CW), BF)
    mesh = plsc.VectorSubcoreMesh(core_axis_name="c", subcore_axis_name="s")
    k = pl.kernel(
        _agg_body,
        out_type=(chunk,) * NCHUNK,
        mesh=mesh,
        scratch_types=(
            [pltpu.VMEM((NBLK, EBLK), jnp.int32)] * 2
            + [pltpu.VMEM((EBLK, CW), BF)] * RING
            + [pltpu.VMEM((ZBUF, CW), BF),
               pltpu.VMEM_SHARED((ACC_ROWS, CW), BF)]
            + [pltpu.SemaphoreType.DMA] * (2 * RING + 1)
        ),
        compiler_params=pltpu.CompilerParams(use_tc_tiling_on_sc=False),
    )
    return k(src_r, dst_r, *hs)


def _pre_body(x_ref, w_ref, b_ref, *outs):
    acc = jnp.dot(x_ref[...], w_ref[...],
                  preferred_element_type=jnp.float32) + b_ref[...]
    for c, o in enumerate(outs):
        o[...] = acc[:, c * CW:(c + 1) * CW].astype(BF)


def _pre(x, W0, b0):
    nf = x.shape[1]
    chunk = jax.ShapeDtypeStruct((N_NODES, CW), BF)
    return pl.pallas_call(
        _pre_body,
        grid=(NROWBLK,),
        in_specs=[
            pl.BlockSpec((ROWBLK, nf), lambda i: (i, 0)),
            pl.BlockSpec((nf, NHID), lambda i: (0, 0)),
            pl.BlockSpec((1, NHID), lambda i: (0, 0)),
        ],
        out_specs=[pl.BlockSpec((ROWBLK, CW), lambda i: (i, 0))] * NCHUNK,
        out_shape=(chunk,) * NCHUNK,
    )(x, W0, b0.reshape(1, NHID))


def _mlp_body(*refs):
    hs = refs[:NCHUNK]
    as_ = refs[NCHUNK:2 * NCHUNK]
    wa_ref, ba_ref, wb_ref, bb_ref = refs[2 * NCHUNK:2 * NCHUNK + 4]
    outs = refs[2 * NCHUNK + 4:]
    z = jnp.concatenate(
        [hs[c][...].astype(jnp.float32) + as_[c][...].astype(jnp.float32)
         for c in range(NCHUNK)], axis=1)
    acc = jnp.dot(z, wa_ref[...], preferred_element_type=jnp.float32) \
        + ba_ref[...]
    u = jnp.maximum(acc, 0.0)
    v = jnp.dot(u, wb_ref[...], preferred_element_type=jnp.float32) \
        + bb_ref[...]
    v = jnp.maximum(v, 0.0)
    for c, o in enumerate(outs):
        o[...] = v[:, c * CW:(c + 1) * CW].astype(BF)


def _mlp(hs, aggs, Wa_i, ba_i, Wb_i, bb_i):
    chunk = jax.ShapeDtypeStruct((N_NODES, CW), BF)
    cb = pl.BlockSpec((ROWBLK, CW), lambda i: (i, 0))
    return pl.pallas_call(
        _mlp_body,
        grid=(NROWBLK,),
        in_specs=[cb] * NCHUNK + [cb] * NCHUNK + [
            pl.BlockSpec((NHID, NHID), lambda i: (0, 0)),
            pl.BlockSpec((1, NHID), lambda i: (0, 0)),
            pl.BlockSpec((NHID, NHID), lambda i: (0, 0)),
            pl.BlockSpec((1, NHID), lambda i: (0, 0)),
        ],
        out_specs=[cb] * NCHUNK,
        out_shape=(chunk,) * NCHUNK,
    )(*hs, *aggs, Wa_i, ba_i.reshape(1, NHID), Wb_i, bb_i.reshape(1, NHID))


def _pool_body(*refs):
    hs = refs[:NCHUNK]
    batch_ref, wp_ref, bp_ref, wr_ref, br_ref = refs[NCHUNK:NCHUNK + 5]
    out_ref, g_acc = refs[NCHUNK + 5:]
    i = pl.program_id(0)

    @pl.when(i == 0)
    def _():
        g_acc[...] = jnp.zeros((N_GRAPHS, NHID), jnp.float32)

    b = batch_ref[0, 0, :]
    onehot = (b[:, None] == lax.broadcasted_iota(jnp.int32, (1, N_GRAPHS), 1)
              ).astype(jnp.float32)
    z = jnp.concatenate([hs[c][...].astype(jnp.float32)
                         for c in range(NCHUNK)], axis=1)
    g_acc[...] += lax.dot_general(onehot, z, (((0,), (0,)), ((), ())),
                                  preferred_element_type=jnp.float32)

    @pl.when(i == NROWBLK - 1)
    def _():
        g = g_acc[...]
        g = jnp.maximum(jnp.dot(g, wp_ref[...],
                                preferred_element_type=jnp.float32)
                        + bp_ref[...], 0.0)
        o = jnp.dot(g, wr_ref[...], preferred_element_type=jnp.float32) \
            + br_ref[...]
        m = jnp.max(o, axis=1, keepdims=True)
        lse = jnp.log(jnp.sum(jnp.exp(o - m), axis=1, keepdims=True)) + m
        out_ref[...] = o - lse


def _pool(hs, batch, Wp, bp, Wr, br):
    nclass = Wr.shape[1]
    cb = pl.BlockSpec((ROWBLK, CW), lambda i: (i, 0))
    return pl.pallas_call(
        _pool_body,
        grid=(NROWBLK,),
        in_specs=[cb] * NCHUNK + [
            pl.BlockSpec((1, 1, ROWBLK), lambda i: (i, 0, 0)),
            pl.BlockSpec((NHID, NHID), lambda i: (0, 0)),
            pl.BlockSpec((1, NHID), lambda i: (0, 0)),
            pl.BlockSpec((NHID, nclass), lambda i: (0, 0)),
            pl.BlockSpec((1, nclass), lambda i: (0, 0)),
        ],
        out_specs=pl.BlockSpec((N_GRAPHS, nclass), lambda i: (0, 0)),
        out_shape=jax.ShapeDtypeStruct((N_GRAPHS, nclass), jnp.float32),
        scratch_shapes=[pltpu.VMEM((N_GRAPHS, NHID), jnp.float32)],
    )(*hs, batch.reshape(NROWBLK, 1, ROWBLK).astype(jnp.int32),
      Wp, bp.reshape(1, NHID), Wr, br.reshape(1, nclass))


def kernel(x, edge_index, batch, W0, b0, Wa, ba, Wb, bb, Wp, bp, Wr, br):
    eps = N_EDGES // NSUB
    pad = EPS_PAD - eps
    src = jnp.pad(edge_index[0].astype(jnp.int32).reshape(NSUB, eps),
                  ((0, 0), (0, pad))).reshape(NSUB, NBLK, EBLK)
    dst = jnp.pad(edge_index[1].astype(jnp.int32).reshape(NSUB, eps),
                  ((0, 0), (0, pad)),
                  constant_values=TRASH).reshape(NSUB, NBLK, EBLK)
    hs = _pre(x, W0, b0)

    def layer(carry, w):
        Wa_i, ba_i, Wb_i, bb_i = w
        aggs = _agg_sc(src, dst, carry)
        return _mlp(carry, aggs, Wa_i, ba_i, Wb_i, bb_i), None

    hs, _ = lax.scan(layer, hs, (Wa, ba, Wb, bb))
    return _pool(hs, batch, Wp, bp, Wr, br)


# speedup vs baseline: 4.7279x; 1.0001x over previous
import jax
import jax.numpy as jnp
from jax import lax
from jax.experimental import pallas as pl
from jax.experimental.pallas import tpu as pltpu
from jax.experimental.pallas import tpu_sc as plsc

N_NODES = 10000
N_EDGES = 160000
N_GRAPHS = 64
NCHUNK = 4
CW = 128
NHID = 512
BF = jnp.bfloat16

NSUB = 16
NCORE = 2
CPC = NCHUNK // NCORE
EBLK = 128
RING = 5
EPS_PAD = 10240
NBLK = EPS_PAD // EBLK
TRASH = N_NODES
ACC_ROWS = N_NODES + 16
ZSTRIPE = 1000
NZSUB = N_NODES // ZSTRIPE
ZBUF = 200

ROWBLK = 2000
NROWBLK = N_NODES // ROWBLK


def _agg_body(src_hbm, dst_hbm, *refs):
    hs = refs[:NCHUNK]
    aggs = refs[NCHUNK:2 * NCHUNK]
    rest = refs[2 * NCHUNK:]
    src_v, dst_v = rest[0], rest[1]
    rows = rest[2:2 + RING]
    zero_v, acc_s = rest[2 + RING], rest[3 + RING]
    gsem = rest[4 + RING:4 + 2 * RING]
    ssem = rest[4 + 2 * RING:4 + 3 * RING]
    zsem = rest[4 + 3 * RING]
    cid = lax.axis_index("c")
    sid = lax.axis_index("s")

    pltpu.sync_copy(src_hbm.at[sid], src_v)
    pltpu.sync_copy(dst_hbm.at[sid], dst_v)

    zvec = jnp.zeros((32,), BF)

    @pl.loop(0, ZBUF)
    def _(r):
        @pl.loop(0, CW // 32)
        def _(c):
            zero_v[r, pl.ds(c * 32, 32)] = zvec

    def process(h_hbm, a_hbm):
        for i in range(RING - 1):
            pltpu.async_copy(h_hbm.at[src_v.at[i]], rows[i], gsem[i])

        @pl.when(sid < NZSUB)
        def _():
            @pl.loop(0, ZSTRIPE // ZBUF)
            def _(t):
                pltpu.sync_copy(zero_v,
                                acc_s.at[pl.ds(sid * ZSTRIPE + t * ZBUF,
                                               ZBUF)])

        plsc.subcore_barrier()

        @pl.loop(0, NBLK // RING)
        def _(g):
            for kk in range(RING):
                j = g * RING + kk
                bp = (kk + RING - 1) % RING
                pltpu.make_async_copy(h_hbm.at[src_v.at[j]], rows[kk],
                                      gsem[kk]).wait()
                pltpu.async_copy(rows[kk], acc_s.at[dst_v.at[j]], ssem[kk],
                                 add=True)
                if kk == 0:
                    @pl.when(g >= 1)
                    def _():
                        pltpu.make_async_copy(rows[bp], acc_s.at[dst_v.at[j]],
                                              ssem[bp]).wait()
                    pltpu.async_copy(h_hbm.at[src_v.at[j + RING - 1]],
                                     rows[bp], gsem[bp])
                else:
                    @pl.when(g <= NBLK // RING - 2)
                    def _():
                        pltpu.make_async_copy(rows[bp], acc_s.at[dst_v.at[j]],
                                              ssem[bp]).wait()
                        pltpu.async_copy(h_hbm.at[src_v.at[j + RING - 1]],
                                         rows[bp], gsem[bp])

        for kk in range(RING):
            pltpu.make_async_copy(rows[kk], acc_s.at[dst_v.at[0]],
                                  ssem[kk]).wait()

        plsc.subcore_barrier()

        @pl.when(sid < NZSUB)
        def _():
            pltpu.sync_copy(acc_s.at[pl.ds(sid * ZSTRIPE, ZSTRIPE)],
                            a_hbm.at[pl.ds(sid * ZSTRIPE, ZSTRIPE)])

        plsc.subcore_barrier()

    @pl.when(cid == 0)
    def _():
        for c in range(CPC):
            process(hs[c], aggs[c])

    @pl.when(cid == 1)
    def _():
        for c in range(CPC):
            process(hs[CPC + c], aggs[CPC + c])


def _agg_sc(src_r, dst_r, hs):
    chunk = jax.ShapeDtypeStruct((N_NODES, CW), BF)
    mesh = plsc.VectorSubcoreMesh(core_axis_name="c", subcore_axis_name="s")
    k = pl.kernel(
        _agg_body,
        out_type=(chunk,) * NCHUNK,
        mesh=mesh,
        scratch_types=(
            [pltpu.VMEM((NBLK, EBLK), jnp.int32)] * 2
            + [pltpu.VMEM((EBLK, CW), BF)] * RING
            + [pltpu.VMEM((ZBUF, CW), BF),
               pltpu.VMEM_SHARED((ACC_ROWS, CW), BF)]
            + [pltpu.SemaphoreType.DMA] * (2 * RING + 1)
        ),
        compiler_params=pltpu.CompilerParams(use_tc_tiling_on_sc=False, disable_bounds_checks=True, disable_semaphore_checks=True),
    )
    return k(src_r, dst_r, *hs)


def _pre_body(x_ref, w_ref, b_ref, *outs):
    acc = jnp.dot(x_ref[...], w_ref[...],
                  preferred_element_type=jnp.float32) + b_ref[...]
    for c, o in enumerate(outs):
        o[...] = acc[:, c * CW:(c + 1) * CW].astype(BF)


def _pre(x, W0, b0):
    nf = x.shape[1]
    chunk = jax.ShapeDtypeStruct((N_NODES, CW), BF)
    return pl.pallas_call(
        _pre_body,
        grid=(NROWBLK,),
        in_specs=[
            pl.BlockSpec((ROWBLK, nf), lambda i: (i, 0)),
            pl.BlockSpec((nf, NHID), lambda i: (0, 0)),
            pl.BlockSpec((1, NHID), lambda i: (0, 0)),
        ],
        out_specs=[pl.BlockSpec((ROWBLK, CW), lambda i: (i, 0))] * NCHUNK,
        out_shape=(chunk,) * NCHUNK,
    )(x, W0, b0.reshape(1, NHID))


def _mlp_body(*refs):
    hs = refs[:NCHUNK]
    as_ = refs[NCHUNK:2 * NCHUNK]
    wa_ref, ba_ref, wb_ref, bb_ref = refs[2 * NCHUNK:2 * NCHUNK + 4]
    outs = refs[2 * NCHUNK + 4:]
    z = jnp.concatenate(
        [hs[c][...].astype(jnp.float32) + as_[c][...].astype(jnp.float32)
         for c in range(NCHUNK)], axis=1)
    acc = jnp.dot(z, wa_ref[...], preferred_element_type=jnp.float32) \
        + ba_ref[...]
    u = jnp.maximum(acc, 0.0)
    v = jnp.dot(u, wb_ref[...], preferred_element_type=jnp.float32) \
        + bb_ref[...]
    v = jnp.maximum(v, 0.0)
    for c, o in enumerate(outs):
        o[...] = v[:, c * CW:(c + 1) * CW].astype(BF)


def _mlp(hs, aggs, Wa_i, ba_i, Wb_i, bb_i):
    chunk = jax.ShapeDtypeStruct((N_NODES, CW), BF)
    cb = pl.BlockSpec((ROWBLK, CW), lambda i: (i, 0))
    return pl.pallas_call(
        _mlp_body,
        grid=(NROWBLK,),
        in_specs=[cb] * NCHUNK + [cb] * NCHUNK + [
            pl.BlockSpec((NHID, NHID), lambda i: (0, 0)),
            pl.BlockSpec((1, NHID), lambda i: (0, 0)),
            pl.BlockSpec((NHID, NHID), lambda i: (0, 0)),
            pl.BlockSpec((1, NHID), lambda i: (0, 0)),
        ],
        out_specs=[cb] * NCHUNK,
        out_shape=(chunk,) * NCHUNK,
    )(*hs, *aggs, Wa_i, ba_i.reshape(1, NHID), Wb_i, bb_i.reshape(1, NHID))


def _pool_body(*refs):
    hs = refs[:NCHUNK]
    batch_ref, wp_ref, bp_ref, wr_ref, br_ref = refs[NCHUNK:NCHUNK + 5]
    out_ref, g_acc = refs[NCHUNK + 5:]
    i = pl.program_id(0)

    @pl.when(i == 0)
    def _():
        g_acc[...] = jnp.zeros((N_GRAPHS, NHID), jnp.float32)

    b = batch_ref[0, 0, :]
    onehot = (b[:, None] == lax.broadcasted_iota(jnp.int32, (1, N_GRAPHS), 1)
              ).astype(jnp.float32)
    z = jnp.concatenate([hs[c][...].astype(jnp.float32)
                         for c in range(NCHUNK)], axis=1)
    g_acc[...] += lax.dot_general(onehot, z, (((0,), (0,)), ((), ())),
                                  preferred_element_type=jnp.float32)

    @pl.when(i == NROWBLK - 1)
    def _():
        g = g_acc[...]
        g = jnp.maximum(jnp.dot(g, wp_ref[...],
                                preferred_element_type=jnp.float32)
                        + bp_ref[...], 0.0)
        o = jnp.dot(g, wr_ref[...], preferred_element_type=jnp.float32) \
            + br_ref[...]
        m = jnp.max(o, axis=1, keepdims=True)
        lse = jnp.log(jnp.sum(jnp.exp(o - m), axis=1, keepdims=True)) + m
        out_ref[...] = o - lse


def _pool(hs, batch, Wp, bp, Wr, br):
    nclass = Wr.shape[1]
    cb = pl.BlockSpec((ROWBLK, CW), lambda i: (i, 0))
    return pl.pallas_call(
        _pool_body,
        grid=(NROWBLK,),
        in_specs=[cb] * NCHUNK + [
            pl.BlockSpec((1, 1, ROWBLK), lambda i: (i, 0, 0)),
            pl.BlockSpec((NHID, NHID), lambda i: (0, 0)),
            pl.BlockSpec((1, NHID), lambda i: (0, 0)),
            pl.BlockSpec((NHID, nclass), lambda i: (0, 0)),
            pl.BlockSpec((1, nclass), lambda i: (0, 0)),
        ],
        out_specs=pl.BlockSpec((N_GRAPHS, nclass), lambda i: (0, 0)),
        out_shape=jax.ShapeDtypeStruct((N_GRAPHS, nclass), jnp.float32),
        scratch_shapes=[pltpu.VMEM((N_GRAPHS, NHID), jnp.float32)],
    )(*hs, batch.reshape(NROWBLK, 1, ROWBLK).astype(jnp.int32),
      Wp, bp.reshape(1, NHID), Wr, br.reshape(1, nclass))


def kernel(x, edge_index, batch, W0, b0, Wa, ba, Wb, bb, Wp, bp, Wr, br):
    eps = N_EDGES // NSUB
    pad = EPS_PAD - eps
    src = jnp.pad(edge_index[0].astype(jnp.int32).reshape(NSUB, eps),
                  ((0, 0), (0, pad))).reshape(NSUB, NBLK, EBLK)
    dst = jnp.pad(edge_index[1].astype(jnp.int32).reshape(NSUB, eps),
                  ((0, 0), (0, pad)),
                  constant_values=TRASH).reshape(NSUB, NBLK, EBLK)
    hs = _pre(x, W0, b0)

    def layer(carry, w):
        Wa_i, ba_i, Wb_i, bb_i = w
        aggs = _agg_sc(src, dst, carry)
        return _mlp(carry, aggs, Wa_i, ba_i, Wb_i, bb_i), None

    hs, _ = lax.scan(layer, hs, (Wa, ba, Wb, bb))
    return _pool(hs, batch, Wp, bp, Wr, br)
